# baseline (speedup 1.0000x reference)
import functools

import jax
import jax.numpy as jnp
import numpy as np
from jax import lax
from jax.experimental import pallas as pl
from jax.experimental.pallas import tpu as pltpu
from jax.experimental.pallas import tpu_sc as plsc

L = 16
S = 1
_MININT = np.int32(-2147483648)


def _normalize(th):
    n2 = jnp.sum(th * th, axis=1, keepdims=True)
    return th / jnp.maximum(jnp.sqrt(n2), 1e-12)


def _monotone(v):
    return jnp.where(v < 0, ~v, v ^ _MININT)


def _unmonotone_bits(m):
    return jnp.where(m < 0, m ^ _MININT, ~m)


def _proj_body(x_ref, y_ref, th_ref, xpb_ref, xk_ref, yk_ref):
    th = _normalize(th_ref[...])
    dn = (((1,), (1,)), ((), ()))
    xp = lax.dot_general(th, x_ref[0], dn, preferred_element_type=jnp.float32)
    yp = lax.dot_general(th, y_ref[0], dn, preferred_element_type=jnp.float32)
    xpb = lax.bitcast_convert_type(xp, jnp.int32)
    xpb_ref[0] = xpb
    xk_ref[0] = _monotone(xpb)
    yk_ref[0] = _monotone(lax.bitcast_convert_type(yp, jnp.int32))


def _project(x, y, thetas, bn):
    B, N, D = x.shape
    P = thetas.shape[0]
    grid = (B, N // bn)
    xy_spec = pl.BlockSpec((1, bn, D), lambda b, n: (b, n, 0))
    th_spec = pl.BlockSpec((P, D), lambda b, n: (0, 0))
    out_spec = pl.BlockSpec((1, P, bn), lambda b, n: (b, 0, n))
    shape = jax.ShapeDtypeStruct((B, P, N), jnp.int32)
    return pl.pallas_call(
        _proj_body,
        grid=grid,
        in_specs=[xy_spec, xy_spec, th_spec],
        out_specs=[out_spec, out_spec, out_spec],
        out_shape=[shape, shape, shape],
    )(x, y, thetas)


def _scan_hist(hist, aux1, aux2):
    nhv = hist.shape[0] // L
    n2 = nhv // L
    lane = lax.iota(jnp.int32, L)
    last = lane == (L - 1)

    @plsc.parallel_loop(0, nhv, unroll=8)
    def _(i):
        v = hist[pl.ds(i * L, L)]
        s = jnp.cumsum(v)
        hist[pl.ds(i * L, L)] = s - v
        plsc.store_scatter(aux1, [jnp.full((L,), i, jnp.int32)], s, mask=last)

    @plsc.parallel_loop(0, n2, unroll=4)
    def _(i):
        v = aux1[pl.ds(i * L, L)]
        s = jnp.cumsum(v)
        aux1[pl.ds(i * L, L)] = s - v
        plsc.store_scatter(aux2, [jnp.full((L,), i, jnp.int32)], s, mask=last)

    def c_body(i, carry):
        v = aux2[pl.ds(i * L, L)]
        s = jnp.cumsum(v)
        aux2[pl.ds(i * L, L)] = s - v + carry
        return carry + jnp.sum(v)

    lax.fori_loop(0, n2 // L if n2 > L else 1, c_body, jnp.int32(0))

    @plsc.parallel_loop(0, nhv, unroll=8)
    def _(i):
        b1 = plsc.load_gather(aux1, [jnp.full((L,), i, jnp.int32)])
        b2 = plsc.load_gather(aux2, [jnp.full((L,), i >> 4, jnp.int32)])
        hist[pl.ds(i * L, L)] = hist[pl.ds(i * L, L)] + b1 + b2


def _t_of(o, chunk):
    return (o & (chunk - 1)) * L + lax.shift_right_logical(o, chunk.bit_length() - 1)


def _radix_pass_xy(xk_s, xk_d, xv_s, xv_d, yk_s, yk_d, hx, hy, ax1, ax2, ay1, ay2,
                   shift, chunk, lane, first, last):
    sub = chunk // S

    @plsc.parallel_loop(0, hx.shape[0] // L, unroll=8)
    def _(i):
        hx[pl.ds(i * L, L)] = jnp.zeros((L,), jnp.int32)
        hy[pl.ds(i * L, L)] = jnp.zeros((L,), jnp.int32)

    ones = jnp.ones((L,), jnp.int32)
    lane_s = lane * S

    @plsc.parallel_loop(0, sub, unroll=8)
    def _(q):
        for s in range(S):
            sl = pl.ds((s * sub + q) * L, L)
            dx = lax.shift_right_logical(xk_s[sl], shift) & 255
            plsc.addupdate_scatter(hx, [dx * (L * S) + lane_s + s], ones)
            dy = lax.shift_right_logical(yk_s[sl], shift) & 255
            plsc.addupdate_scatter(hy, [dy * (L * S) + lane_s + s], ones)

    _scan_hist(hx, ax1, ax2)
    _scan_hist(hy, ay1, ay2)

    U = 4

    def load_kv(q):
        out = []
        for s in range(S):
            for u in range(U):
                i = s * sub + q * U + u
                sl = pl.ds(i * L, L)
                kx = xk_s[sl]
                ky = yk_s[sl]
                vx = (lane + i * L) if first else xv_s[sl]
                hix = (lax.shift_right_logical(kx, shift) & 255) * (L * S) + lane_s + s
                hiy = (lax.shift_right_logical(ky, shift) & 255) * (L * S) + lane_s + s
                out.append((kx, vx, ky, hix, hiy))
        return tuple(out)

    def perm_body(q, c):
        n_ = S * U
        ox = [plsc.load_gather(hx, [c[k][3]]) for k in range(n_)]
        oy = [plsc.load_gather(hy, [c[k][4]]) for k in range(n_)]
        nc_ = load_kv(jnp.minimum(q + 1, sub // U - 1))
        for s in range(S):
            for u in range(1, U):
                k = s * U + u
                for j in range(s * U, k):
                    ox[k] = ox[k] + jnp.where(c[k][3] == c[j][3], 1, 0)
                    oy[k] = oy[k] + jnp.where(c[k][4] == c[j][4], 1, 0)
        for k in range(n_):
            plsc.store_scatter(hx, [c[k][3]], ox[k] + 1)
            plsc.store_scatter(hy, [c[k][4]], oy[k] + 1)
        for k in range(n_):
            kx, vx, ky, _, _ = c[k]
            tox = _t_of(ox[k], chunk)
            toy = _t_of(oy[k], chunk)
            if not last:
                plsc.store_scatter(xk_d, [tox], kx)
            plsc.store_scatter(xv_d, [tox], vx)
            plsc.store_scatter(yk_d, [toy], ky)
        return nc_

    lax.fori_loop(0, sub // U, perm_body, load_kv(0))


def _sc_transport_body(
    xk_hbm, yk_hbm, xpb_hbm, out_hbm, k0, k1, v0, v1, y0, y1, hx, hy, ax1, ax2, ay1, ay2
):
    nc = 2
    wid = lax.axis_index("s") * nc + lax.axis_index("c")
    rows = xk_hbm.shape[0]
    n = xk_hbm.shape[1]
    chunk = n // L
    nvec = n // L
    lane = lax.iota(jnp.int32, L)
    rows_per_w = rows // 32

    def row_body(t, c):
        r = wid * rows_per_w + t
        pltpu.sync_copy(xk_hbm.at[r], k0)
        pltpu.sync_copy(yk_hbm.at[r], y0)

        for p in range(4):
            s, d = (k0, k1) if p % 2 == 0 else (k1, k0)
            sv, dv = (v0, v1) if p % 2 == 0 else (v1, v0)
            sy, dy = (y0, y1) if p % 2 == 0 else (y1, y0)
            _radix_pass_xy(s, d, sv, dv, sy, dy, hx, hy, ax1, ax2, ay1, ay2,
                           8 * p, chunk, lane, p == 0, p == 3)

        pltpu.sync_copy(xpb_hbm.at[r], k1)

        @plsc.parallel_loop(0, nvec, unroll=4)
        def _(i):
            sl = pl.ds(i * L, L)
            idx = v0[sl]
            ysf = lax.bitcast_convert_type(_unmonotone_bits(y0[sl]), jnp.float32)
            xpf = lax.bitcast_convert_type(plsc.load_gather(k1, [idx]), jnp.float32)
            plsc.store_scatter(k0, [idx], lax.bitcast_convert_type(ysf - xpf, jnp.int32))

        pltpu.sync_copy(k0, out_hbm.at[r])
        return c

    lax.fori_loop(0, rows_per_w, row_body, 0)


def _sc_transport(xk, yk, xpb):
    R, N = xk.shape
    mesh = plsc.VectorSubcoreMesh(
        core_axis_name="c", subcore_axis_name="s", num_cores=2, num_subcores=16
    )
    nbins = 256 * L * S
    f = pl.kernel(
        _sc_transport_body,
        out_type=jax.ShapeDtypeStruct((R, N), jnp.int32),
        mesh=mesh,
        compiler_params=pltpu.CompilerParams(needs_layout_passes=False),
        scratch_types=[
            pltpu.VMEM((N,), jnp.int32),
            pltpu.VMEM((N,), jnp.int32),
            pltpu.VMEM((N,), jnp.int32),
            pltpu.VMEM((N,), jnp.int32),
            pltpu.VMEM((N,), jnp.int32),
            pltpu.VMEM((N,), jnp.int32),
            pltpu.VMEM((nbins,), jnp.int32),
            pltpu.VMEM((nbins,), jnp.int32),
            pltpu.VMEM((nbins // L,), jnp.int32),
            pltpu.VMEM((max(nbins // L // L, L),), jnp.int32),
            pltpu.VMEM((nbins // L,), jnp.int32),
            pltpu.VMEM((max(nbins // L // L, L),), jnp.int32),
        ],
    )
    return f(xk, yk, xpb)


def _recomb_body(x_ref, diff_ref, th_ref, o_ref, *, inv_p):
    th = _normalize(th_ref[...])
    diff = lax.bitcast_convert_type(diff_ref[0], jnp.float32)
    dn = (((0,), (0,)), ((), ()))
    contrib = lax.dot_general(diff, th, dn, preferred_element_type=jnp.float32)
    o_ref[0] = x_ref[0] + contrib * inv_p


def _recombine(x, diffb, thetas, bn):
    B, N, D = x.shape
    P = thetas.shape[0]
    grid = (B, N // bn)
    return pl.pallas_call(
        functools.partial(_recomb_body, inv_p=1.0 / P),
        grid=grid,
        in_specs=[
            pl.BlockSpec((1, bn, D), lambda b, n: (b, n, 0)),
            pl.BlockSpec((1, P, bn), lambda b, n: (b, 0, n)),
            pl.BlockSpec((P, D), lambda b, n: (0, 0)),
        ],
        out_specs=pl.BlockSpec((1, bn, D), lambda b, n: (b, n, 0)),
        out_shape=jax.ShapeDtypeStruct((B, N, D), jnp.float32),
    )(x, diffb, thetas)


def kernel(x_batch, y_batch, thetas, eps, n_projections):
    B, N, D = x_batch.shape
    P = thetas.shape[0]
    bn = 4096
    xpb, xk, yk = _project(x_batch, y_batch, thetas, bn)
    diffb = _sc_transport(
        xk.reshape(B * P, N), yk.reshape(B * P, N), xpb.reshape(B * P, N)
    )
    return _recombine(x_batch, diffb.reshape(B, P, N), thetas, bn)


# speedup vs baseline: 1.0027x; 1.0027x over previous
import functools

import jax
import jax.numpy as jnp
import numpy as np
from jax import lax
from jax.experimental import pallas as pl
from jax.experimental.pallas import tpu as pltpu
from jax.experimental.pallas import tpu_sc as plsc

L = 16
S = 1
_MININT = np.int32(-2147483648)


def _normalize(th):
    n2 = jnp.sum(th * th, axis=1, keepdims=True)
    return th / jnp.maximum(jnp.sqrt(n2), 1e-12)


def _monotone(v):
    return jnp.where(v < 0, ~v, v ^ _MININT)


def _unmonotone_bits(m):
    return jnp.where(m < 0, m ^ _MININT, ~m)


def _proj_body(x_ref, y_ref, th_ref, xpb_ref, xk_ref, yk_ref):
    th = _normalize(th_ref[...])
    dn = (((1,), (1,)), ((), ()))
    xp = lax.dot_general(th, x_ref[0], dn, preferred_element_type=jnp.float32)
    yp = lax.dot_general(th, y_ref[0], dn, preferred_element_type=jnp.float32)
    xpb = lax.bitcast_convert_type(xp, jnp.int32)
    xpb_ref[0] = xpb
    xk_ref[0] = _monotone(xpb)
    yk_ref[0] = _monotone(lax.bitcast_convert_type(yp, jnp.int32))


def _project(x, y, thetas, bn):
    B, N, D = x.shape
    P = thetas.shape[0]
    grid = (B, N // bn)
    xy_spec = pl.BlockSpec((1, bn, D), lambda b, n: (b, n, 0))
    th_spec = pl.BlockSpec((P, D), lambda b, n: (0, 0))
    out_spec = pl.BlockSpec((1, P, bn), lambda b, n: (b, 0, n))
    shape = jax.ShapeDtypeStruct((B, P, N), jnp.int32)
    return pl.pallas_call(
        _proj_body,
        grid=grid,
        in_specs=[xy_spec, xy_spec, th_spec],
        out_specs=[out_spec, out_spec, out_spec],
        out_shape=[shape, shape, shape],
    )(x, y, thetas)


def _scan_hist(hist, aux1, aux2):
    nhv = hist.shape[0] // L
    n2 = nhv // L
    lane = lax.iota(jnp.int32, L)
    last = lane == (L - 1)

    @plsc.parallel_loop(0, nhv, unroll=8)
    def _(i):
        v = hist[pl.ds(i * L, L)]
        s = jnp.cumsum(v)
        hist[pl.ds(i * L, L)] = s - v
        plsc.store_scatter(aux1, [jnp.full((L,), i, jnp.int32)], s, mask=last)

    @plsc.parallel_loop(0, n2, unroll=4)
    def _(i):
        v = aux1[pl.ds(i * L, L)]
        s = jnp.cumsum(v)
        aux1[pl.ds(i * L, L)] = s - v
        plsc.store_scatter(aux2, [jnp.full((L,), i, jnp.int32)], s, mask=last)

    def c_body(i, carry):
        v = aux2[pl.ds(i * L, L)]
        s = jnp.cumsum(v)
        aux2[pl.ds(i * L, L)] = s - v + carry
        return carry + jnp.sum(v)

    lax.fori_loop(0, n2 // L if n2 > L else 1, c_body, jnp.int32(0))

    @plsc.parallel_loop(0, nhv, unroll=8)
    def _(i):
        b1 = plsc.load_gather(aux1, [jnp.full((L,), i, jnp.int32)])
        b2 = plsc.load_gather(aux2, [jnp.full((L,), i >> 4, jnp.int32)])
        hist[pl.ds(i * L, L)] = hist[pl.ds(i * L, L)] + b1 + b2


def _t_of(o, chunk):
    return (o & (chunk - 1)) * L + lax.shift_right_logical(o, chunk.bit_length() - 1)


def _radix_pass_xy(xk_s, xk_d, xv_s, xv_d, yk_s, yk_d, hx, hy, ax1, ax2, ay1, ay2,
                   shift, chunk, lane, first, last):
    sub = chunk // S

    @plsc.parallel_loop(0, hx.shape[0] // L, unroll=8)
    def _(i):
        hx[pl.ds(i * L, L)] = jnp.zeros((L,), jnp.int32)
        hy[pl.ds(i * L, L)] = jnp.zeros((L,), jnp.int32)

    ones = jnp.ones((L,), jnp.int32)
    lane_s = lane * S

    @plsc.parallel_loop(0, sub, unroll=8)
    def _(q):
        for s in range(S):
            sl = pl.ds((s * sub + q) * L, L)
            dx = lax.shift_right_logical(xk_s[sl], shift) & 255
            plsc.addupdate_scatter(hx, [dx * (L * S) + lane_s + s], ones)
            dy = lax.shift_right_logical(yk_s[sl], shift) & 255
            plsc.addupdate_scatter(hy, [dy * (L * S) + lane_s + s], ones)

    _scan_hist(hx, ax1, ax2)
    _scan_hist(hy, ay1, ay2)

    U = 4

    def load_kv(q):
        out = []
        for s in range(S):
            for u in range(U):
                i = s * sub + q * U + u
                sl = pl.ds(i * L, L)
                kx = xk_s[sl]
                ky = yk_s[sl]
                vx = (lane + i * L) if first else xv_s[sl]
                hix = (lax.shift_right_logical(kx, shift) & 255) * (L * S) + lane_s + s
                hiy = (lax.shift_right_logical(ky, shift) & 255) * (L * S) + lane_s + s
                out.append((kx, vx, ky, hix, hiy))
        return tuple(out)

    def perm_body(q, c):
        n_ = S * U
        ox = [plsc.load_gather(hx, [c[k][3]]) for k in range(n_)]
        oy = [plsc.load_gather(hy, [c[k][4]]) for k in range(n_)]
        nc_ = load_kv(jnp.minimum(q + 1, sub // U - 1))
        for s in range(S):
            for u in range(1, U):
                k = s * U + u
                for j in range(s * U, k):
                    ox[k] = ox[k] + jnp.where(c[k][3] == c[j][3], 1, 0)
                    oy[k] = oy[k] + jnp.where(c[k][4] == c[j][4], 1, 0)
        for k in range(n_):
            plsc.store_scatter(hx, [c[k][3]], ox[k] + 1)
            plsc.store_scatter(hy, [c[k][4]], oy[k] + 1)
        for k in range(n_):
            kx, vx, ky, _, _ = c[k]
            tox = ox[k] if last else _t_of(ox[k], chunk)
            toy = oy[k] if last else _t_of(oy[k], chunk)
            if not last:
                plsc.store_scatter(xk_d, [tox], kx)
            plsc.store_scatter(xv_d, [tox], vx)
            plsc.store_scatter(yk_d, [toy], ky)
        return nc_

    lax.fori_loop(0, sub // U, perm_body, load_kv(0))


def _sc_transport_body(
    xk_hbm, yk_hbm, xpb_hbm, out_hbm, k0, k1, v0, v1, y0, y1, hx, hy, ax1, ax2, ay1, ay2
):
    nc = 2
    wid = lax.axis_index("s") * nc + lax.axis_index("c")
    rows = xk_hbm.shape[0]
    n = xk_hbm.shape[1]
    chunk = n // L
    nvec = n // L
    lane = lax.iota(jnp.int32, L)
    rows_per_w = rows // 32

    def row_body(t, c):
        r = wid * rows_per_w + t
        pltpu.sync_copy(xk_hbm.at[r], k0)
        pltpu.sync_copy(yk_hbm.at[r], y0)

        for p in range(4):
            s, d = (k0, k1) if p % 2 == 0 else (k1, k0)
            sv, dv = (v0, v1) if p % 2 == 0 else (v1, v0)
            sy, dy = (y0, y1) if p % 2 == 0 else (y1, y0)
            _radix_pass_xy(s, d, sv, dv, sy, dy, hx, hy, ax1, ax2, ay1, ay2,
                           8 * p, chunk, lane, p == 0, p == 3)

        pltpu.sync_copy(xpb_hbm.at[r], k1)

        @plsc.parallel_loop(0, nvec, unroll=4)
        def _(i):
            sl = pl.ds(i * L, L)
            idx = v0[sl]
            ysf = lax.bitcast_convert_type(_unmonotone_bits(y0[sl]), jnp.float32)
            xpf = lax.bitcast_convert_type(plsc.load_gather(k1, [idx]), jnp.float32)
            plsc.store_scatter(k0, [idx], lax.bitcast_convert_type(ysf - xpf, jnp.int32))

        pltpu.sync_copy(k0, out_hbm.at[r])
        return c

    lax.fori_loop(0, rows_per_w, row_body, 0)


def _sc_transport(xk, yk, xpb):
    R, N = xk.shape
    mesh = plsc.VectorSubcoreMesh(
        core_axis_name="c", subcore_axis_name="s", num_cores=2, num_subcores=16
    )
    nbins = 256 * L * S
    f = pl.kernel(
        _sc_transport_body,
        out_type=jax.ShapeDtypeStruct((R, N), jnp.int32),
        mesh=mesh,
        compiler_params=pltpu.CompilerParams(needs_layout_passes=False),
        scratch_types=[
            pltpu.VMEM((N,), jnp.int32),
            pltpu.VMEM((N,), jnp.int32),
            pltpu.VMEM((N,), jnp.int32),
            pltpu.VMEM((N,), jnp.int32),
            pltpu.VMEM((N,), jnp.int32),
            pltpu.VMEM((N,), jnp.int32),
            pltpu.VMEM((nbins,), jnp.int32),
            pltpu.VMEM((nbins,), jnp.int32),
            pltpu.VMEM((nbins // L,), jnp.int32),
            pltpu.VMEM((max(nbins // L // L, L),), jnp.int32),
            pltpu.VMEM((nbins // L,), jnp.int32),
            pltpu.VMEM((max(nbins // L // L, L),), jnp.int32),
        ],
    )
    return f(xk, yk, xpb)


def _recomb_body(x_ref, diff_ref, th_ref, o_ref, *, inv_p):
    th = _normalize(th_ref[...])
    diff = lax.bitcast_convert_type(diff_ref[0], jnp.float32)
    dn = (((0,), (0,)), ((), ()))
    contrib = lax.dot_general(diff, th, dn, preferred_element_type=jnp.float32)
    o_ref[0] = x_ref[0] + contrib * inv_p


def _recombine(x, diffb, thetas, bn):
    B, N, D = x.shape
    P = thetas.shape[0]
    grid = (B, N // bn)
    return pl.pallas_call(
        functools.partial(_recomb_body, inv_p=1.0 / P),
        grid=grid,
        in_specs=[
            pl.BlockSpec((1, bn, D), lambda b, n: (b, n, 0)),
            pl.BlockSpec((1, P, bn), lambda b, n: (b, 0, n)),
            pl.BlockSpec((P, D), lambda b, n: (0, 0)),
        ],
        out_specs=pl.BlockSpec((1, bn, D), lambda b, n: (b, n, 0)),
        out_shape=jax.ShapeDtypeStruct((B, N, D), jnp.float32),
    )(x, diffb, thetas)


def kernel(x_batch, y_batch, thetas, eps, n_projections):
    B, N, D = x_batch.shape
    P = thetas.shape[0]
    bn = 4096
    xpb, xk, yk = _project(x_batch, y_batch, thetas, bn)
    diffb = _sc_transport(
        xk.reshape(B * P, N), yk.reshape(B * P, N), xpb.reshape(B * P, N)
    )
    return _recombine(x_batch, diffb.reshape(B, P, N), thetas, bn)


# speedup vs baseline: 1.0266x; 1.0238x over previous
import functools

import jax
import jax.numpy as jnp
import numpy as np
from jax import lax
from jax.experimental import pallas as pl
from jax.experimental.pallas import tpu as pltpu
from jax.experimental.pallas import tpu_sc as plsc

L = 16
S = 1
_MININT = np.int32(-2147483648)


def _normalize(th):
    n2 = jnp.sum(th * th, axis=1, keepdims=True)
    return th / jnp.maximum(jnp.sqrt(n2), 1e-12)


def _monotone(v):
    return jnp.where(v < 0, ~v, v ^ _MININT)


def _unmonotone_bits(m):
    return jnp.where(m < 0, m ^ _MININT, ~m)


def _proj_body(x_ref, y_ref, th_ref, xpb_ref, xk_ref, yk_ref):
    th = _normalize(th_ref[...])
    dn = (((1,), (1,)), ((), ()))
    xp = lax.dot_general(th, x_ref[0], dn, preferred_element_type=jnp.float32)
    yp = lax.dot_general(th, y_ref[0], dn, preferred_element_type=jnp.float32)
    xpb = lax.bitcast_convert_type(xp, jnp.int32)
    xpb_ref[0] = xpb
    xk_ref[0] = _monotone(xpb)
    yk_ref[0] = _monotone(lax.bitcast_convert_type(yp, jnp.int32))


def _project(x, y, thetas, bn):
    B, N, D = x.shape
    P = thetas.shape[0]
    grid = (B, N // bn)
    xy_spec = pl.BlockSpec((1, bn, D), lambda b, n: (b, n, 0))
    th_spec = pl.BlockSpec((P, D), lambda b, n: (0, 0))
    out_spec = pl.BlockSpec((1, P, bn), lambda b, n: (b, 0, n))
    shape = jax.ShapeDtypeStruct((B, P, N), jnp.int32)
    return pl.pallas_call(
        _proj_body,
        grid=grid,
        in_specs=[xy_spec, xy_spec, th_spec],
        out_specs=[out_spec, out_spec, out_spec],
        out_shape=[shape, shape, shape],
    )(x, y, thetas)


def _scan_hist(hist, aux1, aux2):
    nhv = hist.shape[0] // L
    n2 = nhv // L
    lane = lax.iota(jnp.int32, L)
    last = lane == (L - 1)

    @plsc.parallel_loop(0, nhv, unroll=8)
    def _(i):
        v = hist[pl.ds(i * L, L)]
        s = jnp.cumsum(v)
        hist[pl.ds(i * L, L)] = s - v
        plsc.store_scatter(aux1, [jnp.full((L,), i, jnp.int32)], s, mask=last)

    @plsc.parallel_loop(0, n2, unroll=4)
    def _(i):
        v = aux1[pl.ds(i * L, L)]
        s = jnp.cumsum(v)
        aux1[pl.ds(i * L, L)] = s - v
        plsc.store_scatter(aux2, [jnp.full((L,), i, jnp.int32)], s, mask=last)

    def c_body(i, carry):
        v = aux2[pl.ds(i * L, L)]
        s = jnp.cumsum(v)
        aux2[pl.ds(i * L, L)] = s - v + carry
        return carry + jnp.sum(v)

    lax.fori_loop(0, n2 // L if n2 > L else 1, c_body, jnp.int32(0))

    @plsc.parallel_loop(0, nhv, unroll=8)
    def _(i):
        b1 = plsc.load_gather(aux1, [jnp.full((L,), i, jnp.int32)])
        b2 = plsc.load_gather(aux2, [jnp.full((L,), i >> 4, jnp.int32)])
        hist[pl.ds(i * L, L)] = hist[pl.ds(i * L, L)] + b1 + b2


def _t_of(o, chunk):
    return (o & (chunk - 1)) * L + lax.shift_right_logical(o, chunk.bit_length() - 1)


def _radix_pass_xy(xk_s, xk_d, xv_s, xv_d, yk_s, yk_d, hx, hy, ax1, ax2, ay1, ay2,
                   shift, chunk, lane, first, last):
    sub = chunk // S

    @plsc.parallel_loop(0, hx.shape[0] // L, unroll=8)
    def _(i):
        hx[pl.ds(i * L, L)] = jnp.zeros((L,), jnp.int32)
        hy[pl.ds(i * L, L)] = jnp.zeros((L,), jnp.int32)

    ones = jnp.ones((L,), jnp.int32)
    lane_s = lane * S

    @plsc.parallel_loop(0, sub, unroll=8)
    def _(q):
        for s in range(S):
            sl = pl.ds((s * sub + q) * L, L)
            dx = lax.shift_right_logical(xk_s[sl], shift) & 255
            plsc.addupdate_scatter(hx, [dx * (L * S) + lane_s + s], ones)
            dy = lax.shift_right_logical(yk_s[sl], shift) & 255
            plsc.addupdate_scatter(hy, [dy * (L * S) + lane_s + s], ones)

    _scan_hist(hx, ax1, ax2)
    _scan_hist(hy, ay1, ay2)

    U = 4

    def load_kv(q):
        out = []
        for s in range(S):
            for u in range(U):
                i = s * sub + q * U + u
                sl = pl.ds(i * L, L)
                kx = xk_s[sl]
                ky = yk_s[sl]
                vx = (lane + i * L) if first else xv_s[sl]
                hix = (lax.shift_right_logical(kx, shift) & 255) * (L * S) + lane_s + s
                hiy = (lax.shift_right_logical(ky, shift) & 255) * (L * S) + lane_s + s
                out.append((kx, vx, ky, hix, hiy))
        return tuple(out)

    def perm_body(q, c):
        n_ = S * U
        ox = [plsc.load_gather(hx, [c[k][3]]) for k in range(n_)]
        oy = [plsc.load_gather(hy, [c[k][4]]) for k in range(n_)]
        nc_ = load_kv(jnp.minimum(q + 1, sub // U - 1))
        for s in range(S):
            for u in range(1, U):
                k = s * U + u
                for j in range(s * U, k):
                    ox[k] = ox[k] + jnp.where(c[k][3] == c[j][3], 1, 0)
                    oy[k] = oy[k] + jnp.where(c[k][4] == c[j][4], 1, 0)
        for k in range(n_):
            plsc.store_scatter(hx, [c[k][3]], ox[k] + 1)
            plsc.store_scatter(hy, [c[k][4]], oy[k] + 1)
        for k in range(n_):
            kx, vx, ky, _, _ = c[k]
            tox = ox[k] if last else _t_of(ox[k], chunk)
            toy = oy[k] if last else _t_of(oy[k], chunk)
            if not last:
                plsc.store_scatter(xk_d, [tox], kx)
            plsc.store_scatter(xv_d, [tox], vx)
            plsc.store_scatter(yk_d, [toy], ky)
        return nc_

    lax.fori_loop(0, sub // U, perm_body, load_kv(0))


def _sc_transport_body(
    xk_hbm, yk_hbm, xpb_hbm, out_hbm, k0, k1, v0, v1, y0, y1, hx, hy, ax1, ax2, ay1, ay2
):
    nc = 2
    wid = lax.axis_index("s") * nc + lax.axis_index("c")
    rows = xk_hbm.shape[0]
    n = xk_hbm.shape[1]
    chunk = n // L
    nvec = n // L
    lane = lax.iota(jnp.int32, L)
    rows_per_w = rows // 32

    def row_body(t, c):
        r = wid * rows_per_w + t
        pltpu.sync_copy(xk_hbm.at[r], k0)
        pltpu.sync_copy(yk_hbm.at[r], y0)

        for p in range(4):
            s, d = (k0, k1) if p % 2 == 0 else (k1, k0)
            sv, dv = (v0, v1) if p % 2 == 0 else (v1, v0)
            sy, dy = (y0, y1) if p % 2 == 0 else (y1, y0)
            _radix_pass_xy(s, d, sv, dv, sy, dy, hx, hy, ax1, ax2, ay1, ay2,
                           8 * p, chunk, lane, p == 0, p == 3)

        pltpu.sync_copy(xpb_hbm.at[r], k1)

        @plsc.parallel_loop(0, nvec, unroll=4)
        def _(i):
            sl = pl.ds(i * L, L)
            idx = v0[sl]
            ysf = lax.bitcast_convert_type(_unmonotone_bits(y0[sl]), jnp.float32)
            xpf = lax.bitcast_convert_type(plsc.load_gather(k1, [idx]), jnp.float32)
            plsc.store_scatter(k0, [idx], lax.bitcast_convert_type(ysf - xpf, jnp.int32))

        pltpu.sync_copy(k0, out_hbm.at[r])
        return c

    lax.fori_loop(0, rows_per_w, row_body, 0)


def _sc_transport(xk, yk, xpb):
    R, N = xk.shape
    mesh = plsc.VectorSubcoreMesh(
        core_axis_name="c", subcore_axis_name="s", num_cores=2, num_subcores=16
    )
    nbins = 256 * L * S
    f = pl.kernel(
        _sc_transport_body,
        out_type=jax.ShapeDtypeStruct((R, N), jnp.int32),
        mesh=mesh,
        compiler_params=pltpu.CompilerParams(needs_layout_passes=False),
        scratch_types=[
            pltpu.VMEM((N,), jnp.int32),
            pltpu.VMEM((N,), jnp.int32),
            pltpu.VMEM((N,), jnp.int32),
            pltpu.VMEM((N,), jnp.int32),
            pltpu.VMEM((N,), jnp.int32),
            pltpu.VMEM((N,), jnp.int32),
            pltpu.VMEM((nbins,), jnp.int32),
            pltpu.VMEM((nbins,), jnp.int32),
            pltpu.VMEM((nbins // L,), jnp.int32),
            pltpu.VMEM((max(nbins // L // L, L),), jnp.int32),
            pltpu.VMEM((nbins // L,), jnp.int32),
            pltpu.VMEM((max(nbins // L // L, L),), jnp.int32),
        ],
    )
    return f(xk, yk, xpb)


def _recomb_body(x_ref, diff_ref, th_ref, o_ref, *, inv_p):
    th = _normalize(th_ref[...])
    diff = lax.bitcast_convert_type(diff_ref[0], jnp.float32)
    dn = (((0,), (0,)), ((), ()))
    contrib = lax.dot_general(diff, th, dn, preferred_element_type=jnp.float32)
    o_ref[0] = x_ref[0] + contrib * inv_p


def _recombine(x, diffb, thetas, bn):
    B, N, D = x.shape
    P = thetas.shape[0]
    grid = (B, N // bn)
    return pl.pallas_call(
        functools.partial(_recomb_body, inv_p=1.0 / P),
        grid=grid,
        in_specs=[
            pl.BlockSpec((1, bn, D), lambda b, n: (b, n, 0)),
            pl.BlockSpec((1, P, bn), lambda b, n: (b, 0, n)),
            pl.BlockSpec((P, D), lambda b, n: (0, 0)),
        ],
        out_specs=pl.BlockSpec((1, bn, D), lambda b, n: (b, n, 0)),
        out_shape=jax.ShapeDtypeStruct((B, N, D), jnp.float32),
    )(x, diffb, thetas)


def kernel(x_batch, y_batch, thetas, eps, n_projections):
    B, N, D = x_batch.shape
    P = thetas.shape[0]
    bn = 8192
    xpb, xk, yk = _project(x_batch, y_batch, thetas, bn)
    diffb = _sc_transport(
        xk.reshape(B * P, N), yk.reshape(B * P, N), xpb.reshape(B * P, N)
    )
    return _recombine(x_batch, diffb.reshape(B, P, N), thetas, bn)
